# baseline (speedup 1.0000x reference)
import functools

import jax
import jax.numpy as jnp
from jax import lax
from jax.experimental import pallas as pl
from jax.experimental.pallas import tpu as pltpu
from jax.experimental.pallas import tpu_sc as plsc

P = 10000
K = 32
D = 128
OUT = 128

NC = 2
NS = 16
NW = NC * NS
P_PAD = 10240
PTS_W = P_PAD // NW
GRP = 4
NG = PTS_W // GRP
IDX_ROWS_W = PTS_W * K // 128


def _mm_body(x_ref, xs_ref, w_ref, b_ref, z_ref, y_ref):
    w1 = w_ref[:D, :]
    wd = w_ref[D:, :] - w1
    z_ref[...] = jnp.dot(
        x_ref[...], w1, preferred_element_type=jnp.float32
    ).astype(jnp.bfloat16)
    y_ref[...] = (
        jnp.dot(xs_ref[...], wd, preferred_element_type=jnp.float32)
        + b_ref[...]
    )


def _fin_body(m_ref, y_ref, o_ref):
    o_ref[...] = jnp.maximum(
        m_ref[...].astype(jnp.float32) + y_ref[...], 0.0
    ).T


def _sc_body(z_hbm, idx_hbm, m_hbm, idx_v, buf0, buf1, out_v, sem0, sem1):
    c = lax.axis_index("c")
    s = lax.axis_index("s")
    wid = s * NC + c
    base_idx_row = wid * IDX_ROWS_W

    pltpu.sync_copy(idx_hbm.at[pl.ds(base_idx_row, IDX_ROWS_W)], idx_v)

    bufs = (buf0, buf1)
    sems = (sem0, sem1)

    def _gather(g, buf, sem):
        return pltpu.make_async_copy(z_hbm.at[idx_v.at[g]], buf, sem)

    _gather(0, buf0, sem0).start()
    _gather(1, buf1, sem1).start()

    def _compute_group(g, buf):
        for i in range(GRP):
            row0 = i * K
            w0 = [buf[row0, pl.ds(d * 16, 16)] for d in range(4)]
            lo = [lax.bitcast_convert_type(w << 16, jnp.float32) for w in w0]
            hi = [lax.bitcast_convert_type(w, jnp.float32) for w in w0]
            for k in range(1, K):
                for d in range(4):
                    w = buf[row0 + k, pl.ds(d * 16, 16)]
                    lo[d] = jnp.maximum(lo[d], lax.bitcast_convert_type(w << 16, jnp.float32))
                    hi[d] = jnp.maximum(hi[d], lax.bitcast_convert_type(w, jnp.float32))
            for d in range(4):
                wlo = lax.bitcast_convert_type(lo[d], jnp.uint32)
                whi = lax.bitcast_convert_type(hi[d], jnp.uint32)
                out_v[g, i, pl.ds(d * 16, 16)] = (
                    (whi & jnp.uint32(0xFFFF0000)) | (wlo >> 16)
                )

    def _step(it, carry):
        for lane in range(2):
            g = it * 2 + lane
            buf, sem = bufs[lane], sems[lane]
            _gather(g, buf, sem).wait()
            _compute_group(g, buf)

            @pl.when(g + 2 < NG)
            def _():
                _gather(g + 2, buf, sem).start()

        return carry

    lax.fori_loop(0, NG // 2, _step, 0)
    pltpu.sync_copy(out_v, m_hbm.at[wid])


def kernel(x, x_subset, knn_index, W, b):
    xf = x[0]
    xsf = x_subset[0]
    idx_flat = knn_index[0].reshape(P * K)
    idx_pad = jnp.pad(idx_flat, (0, P_PAD * K - P * K))
    idx2d = idx_pad.reshape(P_PAD * K // 128, 128)
    b2d = b.reshape(1, OUT)

    blk = 1000
    grid = P // blk
    z, y = pl.pallas_call(
        _mm_body,
        grid=(grid,),
        in_specs=[
            pl.BlockSpec((blk, D), lambda i: (i, 0)),
            pl.BlockSpec((blk, D), lambda i: (i, 0)),
            pl.BlockSpec((2 * D, OUT), lambda i: (0, 0)),
            pl.BlockSpec((1, OUT), lambda i: (0, 0)),
        ],
        out_specs=[
            pl.BlockSpec((blk, OUT), lambda i: (i, 0)),
            pl.BlockSpec((blk, OUT), lambda i: (i, 0)),
        ],
        out_shape=[
            jax.ShapeDtypeStruct((P, OUT), jnp.bfloat16),
            jax.ShapeDtypeStruct((P, OUT), jnp.float32),
        ],
    )(xf, xsf, W, b2d)

    z32 = lax.bitcast_convert_type(z.reshape(P, OUT // 2, 2), jnp.uint32)

    mesh = plsc.VectorSubcoreMesh(
        core_axis_name="c", subcore_axis_name="s", num_cores=NC, num_subcores=NS
    )
    m = pl.kernel(
        _sc_body,
        out_type=jax.ShapeDtypeStruct((NW, NG, GRP, OUT // 2), jnp.uint32),
        mesh=mesh,
        compiler_params=pltpu.CompilerParams(use_tc_tiling_on_sc=False),
        scratch_types=[
            pltpu.VMEM((IDX_ROWS_W, 128), jnp.int32),
            pltpu.VMEM((GRP * K, OUT // 2), jnp.uint32),
            pltpu.VMEM((GRP * K, OUT // 2), jnp.uint32),
            pltpu.VMEM((NG, GRP, OUT // 2), jnp.uint32),
            pltpu.SemaphoreType.DMA,
            pltpu.SemaphoreType.DMA,
        ],
    )(z32, idx2d)

    out = pl.pallas_call(
        _fin_body,
        out_shape=jax.ShapeDtypeStruct((OUT, P), jnp.float32),
    )(lax.bitcast_convert_type(m, jnp.bfloat16).reshape(P_PAD, OUT)[:P], y)

    return out[None]


# speedup vs baseline: 1.4600x; 1.4600x over previous
import functools

import jax
import jax.numpy as jnp
from jax import lax
from jax.experimental import pallas as pl
from jax.experimental.pallas import tpu as pltpu
from jax.experimental.pallas import tpu_sc as plsc

P = 10000
K = 32
D = 128
OUT = 128

NC = 2
NS = 16
NW = NC * NS
P_PAD = 10240
PTS_W = P_PAD // NW
GRP = 4
NG = PTS_W // GRP
IDX_ROWS_W = PTS_W * K // 128
NBUF = 4


def _mm_body(x_ref, xs_ref, w_ref, b_ref, z_ref, y_ref):
    w1 = w_ref[:D, :]
    wd = w_ref[D:, :] - w1
    z_ref[...] = jnp.dot(x_ref[...], w1, preferred_element_type=jnp.float32)
    y_ref[...] = (
        jnp.dot(xs_ref[...], wd, preferred_element_type=jnp.float32)
        + b_ref[...]
    )


def _fin_body(m_ref, y_ref, o_ref):
    o_ref[...] = jnp.maximum(m_ref[...] + y_ref[...], 0.0).T


def _sc_body(z_hbm, idx_hbm, m_hbm, idx_v, buf0, buf1, buf2, buf3,
             out_v, sem0, sem1, sem2, sem3):
    c = lax.axis_index("c")
    s = lax.axis_index("s")
    wid = s * NC + c
    base_idx_row = wid * IDX_ROWS_W

    pltpu.sync_copy(idx_hbm.at[pl.ds(base_idx_row, IDX_ROWS_W)], idx_v)

    bufs = (buf0, buf1, buf2, buf3)
    sems = (sem0, sem1, sem2, sem3)

    def _gather(g, buf, sem):
        return pltpu.make_async_copy(z_hbm.at[idx_v.at[g]], buf, sem)

    for b in range(NBUF):
        _gather(b, bufs[b], sems[b]).start()

    def _compute_group(lane, buf):
        for i in range(GRP):
            row0 = i * K
            accs = [buf[row0, pl.ds(d * 16, 16)] for d in range(8)]
            for k in range(1, K):
                for d in range(8):
                    accs[d] = jnp.maximum(
                        accs[d], buf[row0 + k, pl.ds(d * 16, 16)]
                    )
            for d in range(8):
                out_v[lane, i, pl.ds(d * 16, 16)] = accs[d]

    def _step(it, carry):
        for lane in range(NBUF):
            g = it * NBUF + lane
            buf, sem = bufs[lane], sems[lane]
            _gather(g, buf, sem).wait()
            _compute_group(lane, buf)

            @pl.when(g + NBUF < NG)
            def _():
                _gather(g + NBUF, buf, sem).start()

        pltpu.sync_copy(out_v, m_hbm.at[wid, pl.ds(it * NBUF, NBUF)])
        return carry

    lax.fori_loop(0, NG // NBUF, _step, 0)


def kernel(x, x_subset, knn_index, W, b):
    xf = x[0]
    xsf = x_subset[0]
    idx_flat = knn_index[0].reshape(P * K)
    idx_pad = jnp.pad(idx_flat, (0, P_PAD * K - P * K))
    idx2d = idx_pad.reshape(P_PAD * K // 128, 128)
    b2d = b.reshape(1, OUT)

    blk = 1000
    grid = P // blk
    z, y = pl.pallas_call(
        _mm_body,
        grid=(grid,),
        in_specs=[
            pl.BlockSpec((blk, D), lambda i: (i, 0)),
            pl.BlockSpec((blk, D), lambda i: (i, 0)),
            pl.BlockSpec((2 * D, OUT), lambda i: (0, 0)),
            pl.BlockSpec((1, OUT), lambda i: (0, 0)),
        ],
        out_specs=[
            pl.BlockSpec((blk, OUT), lambda i: (i, 0)),
            pl.BlockSpec((blk, OUT), lambda i: (i, 0)),
        ],
        out_shape=[
            jax.ShapeDtypeStruct((P, OUT), jnp.float32),
            jax.ShapeDtypeStruct((P, OUT), jnp.float32),
        ],
    )(xf, xsf, W, b2d)

    mesh = plsc.VectorSubcoreMesh(
        core_axis_name="c", subcore_axis_name="s", num_cores=NC, num_subcores=NS
    )
    m = pl.kernel(
        _sc_body,
        out_type=jax.ShapeDtypeStruct((NW, NG, GRP, OUT), jnp.float32),
        mesh=mesh,
        scratch_types=[
            pltpu.VMEM((IDX_ROWS_W, 128), jnp.int32),
            pltpu.VMEM((GRP * K, OUT), jnp.float32),
            pltpu.VMEM((GRP * K, OUT), jnp.float32),
            pltpu.VMEM((GRP * K, OUT), jnp.float32),
            pltpu.VMEM((GRP * K, OUT), jnp.float32),
            pltpu.VMEM((NBUF, GRP, OUT), jnp.float32),
            pltpu.SemaphoreType.DMA,
            pltpu.SemaphoreType.DMA,
            pltpu.SemaphoreType.DMA,
            pltpu.SemaphoreType.DMA,
        ],
    )(z, idx2d)

    out = pl.pallas_call(
        _fin_body,
        out_shape=jax.ShapeDtypeStruct((OUT, P), jnp.float32),
    )(m.reshape(P_PAD, OUT)[:P], y)

    return out[None]


# speedup vs baseline: 5.6167x; 3.8471x over previous
import jax
import jax.numpy as jnp
from jax import lax
from jax.experimental import pallas as pl
from jax.experimental.pallas import tpu as pltpu
from jax.experimental.pallas import tpu_sc as plsc

P = 10000
K = 32
D = 128
OUT = 128
H = OUT // 2

NC = 2
NS = 16
NW = NC * NS
P_PAD = 10240
PTS_W = P_PAD // NW
GRP = 4
NG = PTS_W // GRP
IDX_ROWS_W = PTS_W * K // 128
NBUF = 4


def _pack_body(x_ref, w_ref, z_ref):
    w1 = w_ref[:D, :]
    zr = jnp.dot(x_ref[0], w1, preferred_element_type=jnp.float32)
    zu = lax.bitcast_convert_type(zr, jnp.uint32) + jnp.uint32(0x8000)
    z_ref[...] = (zu[:, D // 2:] & jnp.uint32(0xFFFF0000)) | (
        zu[:, : D // 2] >> 16
    )


def _fin_body(m_ref, xs_ref, w_ref, b_ref, o_ref):
    w1 = w_ref[:D, :]
    wd = w_ref[D:, :] - w1
    y = jnp.dot(xs_ref[0], wd, preferred_element_type=jnp.float32) + b_ref[...]
    mu = m_ref[:P, :]
    m_lo = lax.bitcast_convert_type(mu << 16, jnp.float32)
    m_hi = lax.bitcast_convert_type(mu & jnp.uint32(0xFFFF0000), jnp.float32)
    mf = jnp.concatenate([m_lo, m_hi], axis=1)
    o_ref[...] = jnp.maximum(mf + y, 0.0).T


def _sc_body(z_hbm, idx_hbm, m_hbm, z_sh, idx_v, buf0, buf1, buf2, buf3,
             out_v, sem0, sem1, sem2, sem3):
    c = lax.axis_index("c")
    s = lax.axis_index("s")
    wid = s * NC + c
    base_idx_row = wid * IDX_ROWS_W

    ch = 624

    @pl.when(s < NS - 1)
    def _():
        off = pl.multiple_of(s * ch, 8)
        pltpu.sync_copy(z_hbm.at[pl.ds(off, ch)], z_sh.at[pl.ds(off, ch)])

    @pl.when(s == NS - 1)
    def _():
        pltpu.sync_copy(
            z_hbm.at[pl.ds((NS - 1) * ch, P - (NS - 1) * ch)],
            z_sh.at[pl.ds((NS - 1) * ch, P - (NS - 1) * ch)],
        )

    pltpu.sync_copy(idx_hbm.at[pl.ds(base_idx_row, IDX_ROWS_W)], idx_v)
    plsc.subcore_barrier()

    bufs = (buf0, buf1, buf2, buf3)
    sems = (sem0, sem1, sem2, sem3)

    def _gather(g, buf, sem):
        return pltpu.make_async_copy(z_sh.at[idx_v.at[g]], buf, sem)

    for b in range(NBUF):
        _gather(b, bufs[b], sems[b]).start()

    def _compute_group(lane, buf):
        for i in range(GRP):
            row0 = i * K
            w0 = [buf[row0, pl.ds(d * 16, 16)] for d in range(4)]
            alo = [lax.bitcast_convert_type(w << 16, jnp.float32) for w in w0]
            ahi = [lax.bitcast_convert_type(w, jnp.float32) for w in w0]
            for k in range(1, K):
                for d in range(4):
                    w = buf[row0 + k, pl.ds(d * 16, 16)]
                    alo[d] = jnp.maximum(
                        alo[d], lax.bitcast_convert_type(w << 16, jnp.float32)
                    )
                    ahi[d] = jnp.maximum(
                        ahi[d], lax.bitcast_convert_type(w, jnp.float32)
                    )
            for d in range(4):
                blo = lax.bitcast_convert_type(alo[d], jnp.uint32)
                bhi = lax.bitcast_convert_type(ahi[d], jnp.uint32)
                out_v[lane * GRP + i, pl.ds(d * 16, 16)] = (
                    (bhi & jnp.uint32(0xFFFF0000)) | (blo >> 16)
                )

    def _step(it, carry):
        for lane in range(NBUF):
            g = it * NBUF + lane
            buf, sem = bufs[lane], sems[lane]
            _gather(g, buf, sem).wait()
            _compute_group(lane, buf)

            @pl.when(g + NBUF < NG)
            def _():
                _gather(g + NBUF, buf, sem).start()

        pltpu.sync_copy(
            out_v,
            m_hbm.at[pl.ds(wid * PTS_W + it * (NBUF * GRP), NBUF * GRP)],
        )
        return carry

    lax.fori_loop(0, NG // NBUF, _step, 0)


def kernel(x, x_subset, knn_index, W, b):
    idx_flat = knn_index[0].reshape(P * K)
    idx_pad = jnp.pad(idx_flat, (0, P_PAD * K - P * K))
    idx2d = idx_pad.reshape(P_PAD * K // 128, 128)
    b2d = b.reshape(1, OUT)

    blk = 1000
    grid = P // blk
    z32 = pl.pallas_call(
        _pack_body,
        grid=(grid,),
        in_specs=[
            pl.BlockSpec((1, blk, D), lambda i: (0, i, 0)),
            pl.BlockSpec((2 * D, OUT), lambda i: (0, 0)),
        ],
        out_specs=pl.BlockSpec((blk, H), lambda i: (i, 0)),
        out_shape=jax.ShapeDtypeStruct((P, H), jnp.uint32),
    )(x, W)

    mesh = plsc.VectorSubcoreMesh(
        core_axis_name="c", subcore_axis_name="s", num_cores=NC, num_subcores=NS
    )
    m = pl.kernel(
        _sc_body,
        out_type=jax.ShapeDtypeStruct((P_PAD, H), jnp.uint32),
        mesh=mesh,
        compiler_params=pltpu.CompilerParams(use_tc_tiling_on_sc=False),
        scratch_types=[
            pltpu.VMEM_SHARED((P, H), jnp.uint32),
            pltpu.VMEM((IDX_ROWS_W, 128), jnp.int32),
            pltpu.VMEM((GRP * K, H), jnp.uint32),
            pltpu.VMEM((GRP * K, H), jnp.uint32),
            pltpu.VMEM((GRP * K, H), jnp.uint32),
            pltpu.VMEM((GRP * K, H), jnp.uint32),
            pltpu.VMEM((NBUF * GRP, H), jnp.uint32),
            pltpu.SemaphoreType.DMA,
            pltpu.SemaphoreType.DMA,
            pltpu.SemaphoreType.DMA,
            pltpu.SemaphoreType.DMA,
        ],
    )(z32, idx2d)

    out = pl.pallas_call(
        _fin_body,
        out_shape=jax.ShapeDtypeStruct((OUT, P), jnp.float32),
    )(m, x_subset, W, b2d)

    return out[None]


# speedup vs baseline: 5.6563x; 1.0071x over previous
import jax
import jax.numpy as jnp
from jax import lax
from jax.experimental import pallas as pl
from jax.experimental.pallas import tpu as pltpu
from jax.experimental.pallas import tpu_sc as plsc

P = 10000
K = 32
D = 128
OUT = 128
H = OUT // 2

NC = 2
NS = 16
NW = NC * NS
P_PAD = 10240
PTS_W = P_PAD // NW
GRP = 4
NG = PTS_W // GRP
IDX_ROWS_W = PTS_W * K // 128
NBUF = 4


def _pack_body(x_ref, w_ref, z_ref):
    w1 = w_ref[:D, :]
    zr = jnp.dot(x_ref[0], w1, preferred_element_type=jnp.float32)
    zu = lax.bitcast_convert_type(zr, jnp.uint32) + jnp.uint32(0x8000)
    z_ref[...] = (zu[:, D // 2:] & jnp.uint32(0xFFFF0000)) | (
        zu[:, : D // 2] >> 16
    )


def _fin_body(m_ref, xs_ref, w_ref, b_ref, o_ref):
    w1 = w_ref[:D, :]
    wd = w_ref[D:, :] - w1
    y = jnp.dot(xs_ref[0], wd, preferred_element_type=jnp.float32) + b_ref[...]
    mu = m_ref[:P, :]
    m_lo = lax.bitcast_convert_type(mu << 16, jnp.float32)
    m_hi = lax.bitcast_convert_type(mu & jnp.uint32(0xFFFF0000), jnp.float32)
    mf = jnp.concatenate([m_lo, m_hi], axis=1)
    o_ref[0] = jnp.maximum(mf + y, 0.0).T


def _sc_body(z_hbm, idx_hbm, m_hbm, z_sh, idx_v, buf0, buf1, buf2, buf3,
             out_v, sem0, sem1, sem2, sem3):
    c = lax.axis_index("c")
    s = lax.axis_index("s")
    wid = s * NC + c
    base_idx_row = wid * IDX_ROWS_W

    ch = 624

    @pl.when(s < NS - 1)
    def _():
        off = pl.multiple_of(s * ch, 8)
        pltpu.sync_copy(z_hbm.at[pl.ds(off, ch)], z_sh.at[pl.ds(off, ch)])

    @pl.when(s == NS - 1)
    def _():
        pltpu.sync_copy(
            z_hbm.at[pl.ds((NS - 1) * ch, P - (NS - 1) * ch)],
            z_sh.at[pl.ds((NS - 1) * ch, P - (NS - 1) * ch)],
        )

    pltpu.sync_copy(idx_hbm.at[pl.ds(base_idx_row, IDX_ROWS_W)], idx_v)
    plsc.subcore_barrier()

    bufs = (buf0, buf1, buf2, buf3)
    sems = (sem0, sem1, sem2, sem3)

    def _gather(g, buf, sem):
        return pltpu.make_async_copy(z_sh.at[idx_v.at[g]], buf, sem)

    for b in range(NBUF):
        _gather(b, bufs[b], sems[b]).start()

    def _compute_group(lane, buf):
        for i in range(GRP):
            row0 = i * K
            w0 = [buf[row0, pl.ds(d * 16, 16)] for d in range(4)]
            alo = [lax.bitcast_convert_type(w << 16, jnp.float32) for w in w0]
            ahi = [lax.bitcast_convert_type(w, jnp.float32) for w in w0]
            for k in range(1, K):
                for d in range(4):
                    w = buf[row0 + k, pl.ds(d * 16, 16)]
                    alo[d] = jnp.maximum(
                        alo[d], lax.bitcast_convert_type(w << 16, jnp.float32)
                    )
                    ahi[d] = jnp.maximum(
                        ahi[d], lax.bitcast_convert_type(w, jnp.float32)
                    )
            for d in range(4):
                blo = lax.bitcast_convert_type(alo[d], jnp.uint32)
                bhi = lax.bitcast_convert_type(ahi[d], jnp.uint32)
                out_v[lane * GRP + i, pl.ds(d * 16, 16)] = (
                    (bhi & jnp.uint32(0xFFFF0000)) | (blo >> 16)
                )

    def _step(it, carry):
        for lane in range(NBUF):
            g = it * NBUF + lane
            buf, sem = bufs[lane], sems[lane]
            _gather(g, buf, sem).wait()
            _compute_group(lane, buf)

            @pl.when(g + NBUF < NG)
            def _():
                _gather(g + NBUF, buf, sem).start()

        pltpu.sync_copy(
            out_v,
            m_hbm.at[pl.ds(wid * PTS_W + it * (NBUF * GRP), NBUF * GRP)],
        )
        return carry

    lax.fori_loop(0, NG // NBUF, _step, 0)


def kernel(x, x_subset, knn_index, W, b):
    idx_flat = knn_index[0].reshape(P * K)
    idx_pad = jnp.pad(idx_flat, (0, P_PAD * K - P * K))
    idx2d = idx_pad.reshape(P_PAD * K // 128, 128)
    b2d = b.reshape(1, OUT)

    blk = 2000
    grid = P // blk
    z32 = pl.pallas_call(
        _pack_body,
        grid=(grid,),
        in_specs=[
            pl.BlockSpec((1, blk, D), lambda i: (0, i, 0)),
            pl.BlockSpec((2 * D, OUT), lambda i: (0, 0)),
        ],
        out_specs=pl.BlockSpec((blk, H), lambda i: (i, 0)),
        out_shape=jax.ShapeDtypeStruct((P, H), jnp.uint32),
    )(x, W)

    mesh = plsc.VectorSubcoreMesh(
        core_axis_name="c", subcore_axis_name="s", num_cores=NC, num_subcores=NS
    )
    m = pl.kernel(
        _sc_body,
        out_type=jax.ShapeDtypeStruct((P_PAD, H), jnp.uint32),
        mesh=mesh,
        compiler_params=pltpu.CompilerParams(use_tc_tiling_on_sc=False),
        scratch_types=[
            pltpu.VMEM_SHARED((P, H), jnp.uint32),
            pltpu.VMEM((IDX_ROWS_W, 128), jnp.int32),
            pltpu.VMEM((GRP * K, H), jnp.uint32),
            pltpu.VMEM((GRP * K, H), jnp.uint32),
            pltpu.VMEM((GRP * K, H), jnp.uint32),
            pltpu.VMEM((GRP * K, H), jnp.uint32),
            pltpu.VMEM((NBUF * GRP, H), jnp.uint32),
            pltpu.SemaphoreType.DMA,
            pltpu.SemaphoreType.DMA,
            pltpu.SemaphoreType.DMA,
            pltpu.SemaphoreType.DMA,
        ],
    )(z32, idx2d)

    out = pl.pallas_call(
        _fin_body,
        out_shape=jax.ShapeDtypeStruct((1, OUT, P), jnp.float32),
    )(m, x_subset, W, b2d)

    return out


# speedup vs baseline: 7.7315x; 1.3669x over previous
import jax
import jax.numpy as jnp
from jax import lax
from jax.experimental import pallas as pl
from jax.experimental.pallas import tpu as pltpu
from jax.experimental.pallas import tpu_sc as plsc

P = 10000
K = 32
D = 128
OUT = 128
H = OUT // 2

NC = 2
NS = 16
NW = NC * NS
P_PAD = 10240
PTS_W = P_PAD // NW
GRP = 4
NG = PTS_W // GRP
IDX_ROWS_W = PTS_W * K // 128
NBUF = 8


def _pack_body(x_ref, w_ref, z_ref):
    w1 = w_ref[:D, :]
    zr = jnp.dot(x_ref[0], w1, preferred_element_type=jnp.float32)
    zu = lax.bitcast_convert_type(zr, jnp.uint32) + jnp.uint32(0x8000)
    z_ref[...] = (zu[:, D // 2:] & jnp.uint32(0xFFFF0000)) | (
        zu[:, : D // 2] >> 16
    )


def _fin_body(m_ref, xs_ref, w_ref, b_ref, o_ref):
    w1 = w_ref[:D, :]
    wd = w_ref[D:, :] - w1
    y = jnp.dot(xs_ref[0], wd, preferred_element_type=jnp.float32) + b_ref[...]
    mu = m_ref[:P, :]
    m_lo = lax.bitcast_convert_type(mu << 16, jnp.float32)
    m_hi = lax.bitcast_convert_type(mu & jnp.uint32(0xFFFF0000), jnp.float32)
    mf = jnp.concatenate([m_lo, m_hi], axis=1)
    o_ref[0] = jnp.maximum(mf + y, 0.0).T


def _sc_body(z_hbm, idx_hbm, m_hbm, z_sh, idx_v, buf0, buf1, buf2, buf3,
             buf4, buf5, buf6, buf7, out_v,
             sem0, sem1, sem2, sem3, sem4, sem5, sem6, sem7):
    c = lax.axis_index("c")
    s = lax.axis_index("s")
    wid = s * NC + c

    ch = 624

    @pl.when(s < NS - 1)
    def _():
        off = pl.multiple_of(s * ch, 8)
        pltpu.sync_copy(z_hbm.at[pl.ds(off, ch)], z_sh.at[pl.ds(off, ch)])

    @pl.when(s == NS - 1)
    def _():
        pltpu.sync_copy(
            z_hbm.at[pl.ds((NS - 1) * ch, P - (NS - 1) * ch)],
            z_sh.at[pl.ds((NS - 1) * ch, P - (NS - 1) * ch)],
        )

    @pl.when(wid < NW - 1)
    def _():
        boff = pl.multiple_of(wid * PTS_W, 8)
        pltpu.sync_copy(idx_hbm.at[0].at[pl.ds(boff, PTS_W)], idx_v)

    @pl.when(wid == NW - 1)
    def _():
        pltpu.sync_copy(
            idx_hbm.at[0].at[pl.ds((NW - 1) * PTS_W, P - (NW - 1) * PTS_W)],
            idx_v.at[pl.ds(0, P - (NW - 1) * PTS_W)],
        )

    plsc.subcore_barrier()

    bufs = (buf0, buf1, buf2, buf3, buf4, buf5, buf6, buf7)
    sems = (sem0, sem1, sem2, sem3, sem4, sem5, sem6, sem7)

    def _gather(g, buf, sem):
        return pltpu.make_async_copy(z_sh.at[idx_v.at[g]], buf, sem)

    for bi in range(NBUF):
        _gather(bi, bufs[bi], sems[bi]).start()

    def _compute_point(lane, buf):
        w0 = [buf[0, pl.ds(d * 16, 16)] for d in range(4)]
        alo = [lax.bitcast_convert_type(w << 16, jnp.float32) for w in w0]
        ahi = [lax.bitcast_convert_type(w, jnp.float32) for w in w0]
        for k in range(1, K):
            for d in range(4):
                w = buf[k, pl.ds(d * 16, 16)]
                alo[d] = jnp.maximum(
                    alo[d], lax.bitcast_convert_type(w << 16, jnp.float32)
                )
                ahi[d] = jnp.maximum(
                    ahi[d], lax.bitcast_convert_type(w, jnp.float32)
                )
        for d in range(4):
            blo = lax.bitcast_convert_type(alo[d], jnp.uint32)
            bhi = lax.bitcast_convert_type(ahi[d], jnp.uint32)
            out_v[lane, pl.ds(d * 16, 16)] = (
                (bhi & jnp.uint32(0xFFFF0000)) | (blo >> 16)
            )

    n_it = jnp.where(wid == NW - 1, (P - (NW - 1) * PTS_W) // NBUF,
                     PTS_W // NBUF)
    n_pts = n_it * NBUF

    def _step(it, carry):
        for lane in range(NBUF):
            g = it * NBUF + lane
            buf, sem = bufs[lane], sems[lane]
            _gather(g, buf, sem).wait()
            _compute_point(lane, buf)

            @pl.when(g + NBUF < n_pts)
            def _():
                _gather(g + NBUF, buf, sem).start()

        pltpu.sync_copy(
            out_v, m_hbm.at[pl.ds(wid * PTS_W + it * NBUF, NBUF)]
        )
        return carry

    lax.fori_loop(0, n_it, _step, 0)


def kernel(x, x_subset, knn_index, W, b):
    b2d = b.reshape(1, OUT)

    blk = 2000
    grid = P // blk
    z32 = pl.pallas_call(
        _pack_body,
        grid=(grid,),
        in_specs=[
            pl.BlockSpec((1, blk, D), lambda i: (0, i, 0)),
            pl.BlockSpec((2 * D, OUT), lambda i: (0, 0)),
        ],
        out_specs=pl.BlockSpec((blk, H), lambda i: (i, 0)),
        out_shape=jax.ShapeDtypeStruct((P, H), jnp.uint32),
    )(x, W)

    mesh = plsc.VectorSubcoreMesh(
        core_axis_name="c", subcore_axis_name="s", num_cores=NC, num_subcores=NS
    )
    m = pl.kernel(
        _sc_body,
        out_type=jax.ShapeDtypeStruct((P_PAD, H), jnp.uint32),
        mesh=mesh,
        compiler_params=pltpu.CompilerParams(use_tc_tiling_on_sc=False),
        scratch_types=[
            pltpu.VMEM_SHARED((P, H), jnp.uint32),
            pltpu.VMEM((PTS_W, K), jnp.int32),
        ] + [pltpu.VMEM((K, H), jnp.uint32)] * NBUF + [
            pltpu.VMEM((NBUF, H), jnp.uint32),
        ] + [pltpu.SemaphoreType.DMA] * NBUF,
    )(z32, knn_index)

    out = pl.pallas_call(
        _fin_body,
        out_shape=jax.ShapeDtypeStruct((1, OUT, P), jnp.float32),
    )(m, x_subset, W, b2d)

    return out
